# Initial kernel scaffold; baseline (speedup 1.0000x reference)
import jax
import jax.numpy as jnp
from jax.experimental import pallas as pl


def kernel(adj_matrices, ids, W, b):
    raise NotImplementedError("write your pallas kernel here")


# speedup vs baseline: 478.6774x; 478.6774x over previous
import functools

import numpy as np
import jax
import jax.numpy as jnp
from jax.experimental import pallas as pl
from jax.experimental.pallas import tpu as pltpu

_CH = 16


@functools.lru_cache(maxsize=None)
def _pos_encoding(seq_len: int, d: int):
    pos = np.arange(seq_len, dtype=np.float32)[:, None]
    i = np.arange(0, d, 2, dtype=np.float32)[None, :]
    angle = pos / np.power(10000.0, i / d)
    pe = np.zeros((seq_len, d), dtype=np.float32)
    pe[:, 0::2] = np.sin(angle)
    pe[:, 1::2] = np.cos(angle)
    return jnp.asarray(pe)


def _gcn_body(adj_ref, ids_ref, w_ref, b_ref, pe_ref, out_ref):
    n_q = adj_ref.shape[1]
    embed = w_ref.shape[1]
    h = jnp.dot(ids_ref[...], w_ref[...], preferred_element_type=jnp.float32)
    a = adj_ref[...].astype(jnp.float32)
    deg = jnp.sum(a, axis=1)
    dinv = jnp.where(deg > 0.0, jax.lax.rsqrt(jnp.maximum(deg, 1.0)), 0.0)
    a_t = jnp.transpose(a, (0, 2, 1))
    c = a_t * dinv[:, :, None] * dinv[:, None, :]
    out = jnp.dot(c.reshape(_CH * n_q, n_q), h,
                  preferred_element_type=jnp.float32)
    out = out + b_ref[...]
    out = out.reshape(_CH, n_q, embed) + pe_ref[...][:, None, :]
    out_ref[...] = out.reshape(_CH * n_q, embed)


def kernel(adj_matrices, ids, W, b):
    bsz, n_slices, n_q, _ = adj_matrices.shape
    embed = W.shape[1]
    num_blocks = bsz * n_slices
    assert num_blocks % _CH == 0 and n_slices % _CH == 0
    pe = _pos_encoding(n_slices, embed)
    adj3 = adj_matrices.reshape(num_blocks, n_q, n_q)
    b2 = b.reshape(1, embed)

    grid = (num_blocks // _CH,)
    out_flat = pl.pallas_call(
        _gcn_body,
        grid=grid,
        in_specs=[
            pl.BlockSpec((_CH, n_q, n_q), lambda g: (g, 0, 0)),
            pl.BlockSpec((n_q, embed), lambda g: (0, 0)),
            pl.BlockSpec((embed, embed), lambda g: (0, 0)),
            pl.BlockSpec((1, embed), lambda g: (0, 0)),
            pl.BlockSpec((_CH, embed), lambda g: (g % (n_slices // _CH), 0)),
        ],
        out_specs=pl.BlockSpec((_CH * n_q, embed), lambda g: (g, 0)),
        out_shape=jax.ShapeDtypeStruct((num_blocks * n_q, embed), jnp.float32),
        compiler_params=pltpu.CompilerParams(
            dimension_semantics=("arbitrary",),
        ),
    )(adj3, ids[:n_q], W, b2, pe)
    return out_flat.reshape(bsz, n_slices, n_q, embed)


# speedup vs baseline: 945.3564x; 1.9749x over previous
import functools

import numpy as np
import jax
import jax.numpy as jnp
from jax.experimental import pallas as pl
from jax.experimental.pallas import tpu as pltpu

_CH = 128


@functools.lru_cache(maxsize=None)
def _pos_encoding(seq_len: int, d: int):
    pos = np.arange(seq_len, dtype=np.float32)[:, None]
    i = np.arange(0, d, 2, dtype=np.float32)[None, :]
    angle = pos / np.power(10000.0, i / d)
    pe = np.zeros((seq_len, d), dtype=np.float32)
    pe[:, 0::2] = np.sin(angle)
    pe[:, 1::2] = np.cos(angle)
    return jnp.asarray(pe)


def _gcn_body(adj_ref, ids_ref, w_ref, b_ref, pe_ref, out_ref):
    n_q = adj_ref.shape[1]
    embed = w_ref.shape[1]
    h = jnp.dot(ids_ref[...], w_ref[...], preferred_element_type=jnp.float32)
    a = adj_ref[...].astype(jnp.float32)
    deg = jnp.sum(a, axis=1)
    dinv = jnp.where(deg > 0.0, jax.lax.rsqrt(jnp.maximum(deg, 1.0)), 0.0)
    a_t = jnp.transpose(a, (0, 2, 1))
    c = a_t * dinv[:, :, None] * dinv[:, None, :]
    out = jnp.dot(c.reshape(_CH * n_q, n_q), h,
                  preferred_element_type=jnp.float32)
    out = out + b_ref[...]
    out = out.reshape(_CH, n_q, embed) + pe_ref[...][:, None, :]
    out_ref[...] = out.reshape(_CH * n_q, embed)


def kernel(adj_matrices, ids, W, b):
    bsz, n_slices, n_q, _ = adj_matrices.shape
    embed = W.shape[1]
    num_blocks = bsz * n_slices
    assert num_blocks % _CH == 0 and n_slices % _CH == 0
    pe = _pos_encoding(n_slices, embed)
    adj3 = adj_matrices.reshape(num_blocks, n_q, n_q)
    b2 = b.reshape(1, embed)

    grid = (num_blocks // _CH,)
    out_flat = pl.pallas_call(
        _gcn_body,
        grid=grid,
        in_specs=[
            pl.BlockSpec((_CH, n_q, n_q), lambda g: (g, 0, 0)),
            pl.BlockSpec((n_q, embed), lambda g: (0, 0)),
            pl.BlockSpec((embed, embed), lambda g: (0, 0)),
            pl.BlockSpec((1, embed), lambda g: (0, 0)),
            pl.BlockSpec((_CH, embed), lambda g: (g % (n_slices // _CH), 0)),
        ],
        out_specs=pl.BlockSpec((_CH * n_q, embed), lambda g: (g, 0)),
        out_shape=jax.ShapeDtypeStruct((num_blocks * n_q, embed), jnp.float32),
        compiler_params=pltpu.CompilerParams(
            dimension_semantics=("arbitrary",),
        ),
    )(adj3, ids[:n_q], W, b2, pe)
    return out_flat.reshape(bsz, n_slices, n_q, embed)


# speedup vs baseline: 1110.1418x; 1.1743x over previous
import functools

import numpy as np
import jax
import jax.numpy as jnp
from jax.experimental import pallas as pl
from jax.experimental.pallas import tpu as pltpu

_CH = 128


@functools.lru_cache(maxsize=None)
def _pos_encoding(seq_len: int, d: int):
    pos = np.arange(seq_len, dtype=np.float32)[:, None]
    i = np.arange(0, d, 2, dtype=np.float32)[None, :]
    angle = pos / np.power(10000.0, i / d)
    pe = np.zeros((seq_len, d), dtype=np.float32)
    pe[:, 0::2] = np.sin(angle)
    pe[:, 1::2] = np.cos(angle)
    return jnp.asarray(pe)


def _gcn_body(adj_ref, ids_ref, w_ref, b_ref, pe_ref, out_ref, h_ref):
    n_q = ids_ref.shape[0]
    embed = w_ref.shape[1]

    @pl.when(pl.program_id(0) == 0)
    def _():
        h_ref[...] = jnp.dot(ids_ref[...], w_ref[...],
                             preferred_element_type=jnp.float32)

    h = h_ref[...]
    a = adj_ref[...].astype(jnp.float32).reshape(_CH, n_q, n_q)
    deg = jnp.sum(a, axis=1)
    dinv = jnp.where(deg > 0.0, jax.lax.rsqrt(jnp.maximum(deg, 1.0)), 0.0)
    a_t = jnp.transpose(a, (0, 2, 1))
    c = a_t * dinv[:, :, None] * dinv[:, None, :]
    out = jnp.dot(c.reshape(_CH * n_q, n_q), h,
                  preferred_element_type=jnp.float32)
    out = out + b_ref[...]
    out = out.reshape(_CH, n_q, embed) + pe_ref[...][:, None, :]
    out_ref[...] = out.reshape(_CH * n_q, embed)


def kernel(adj_matrices, ids, W, b):
    bsz, n_slices, n_q, _ = adj_matrices.shape
    embed = W.shape[1]
    num_blocks = bsz * n_slices
    assert num_blocks % _CH == 0 and n_slices % _CH == 0
    pe = _pos_encoding(n_slices, embed)
    adj2 = adj_matrices.reshape(num_blocks, n_q * n_q)
    b2 = b.reshape(1, embed)

    grid = (num_blocks // _CH,)
    out_flat = pl.pallas_call(
        _gcn_body,
        grid=grid,
        in_specs=[
            pl.BlockSpec((_CH, n_q * n_q), lambda g: (g, 0)),
            pl.BlockSpec((n_q, embed), lambda g: (0, 0)),
            pl.BlockSpec((embed, embed), lambda g: (0, 0)),
            pl.BlockSpec((1, embed), lambda g: (0, 0)),
            pl.BlockSpec((_CH, embed), lambda g: (g % (n_slices // _CH), 0)),
        ],
        out_specs=pl.BlockSpec((_CH * n_q, embed), lambda g: (g, 0)),
        out_shape=jax.ShapeDtypeStruct((num_blocks * n_q, embed), jnp.float32),
        scratch_shapes=[pltpu.VMEM((n_q, embed), jnp.float32)],
        compiler_params=pltpu.CompilerParams(
            dimension_semantics=("arbitrary",),
        ),
    )(adj2, ids[:n_q], W, b2, pe)
    return out_flat.reshape(bsz, n_slices, n_q, embed)


# speedup vs baseline: 1246.7612x; 1.1231x over previous
import functools

import numpy as np
import jax
import jax.numpy as jnp
from jax.experimental import pallas as pl
from jax.experimental.pallas import tpu as pltpu

_CH = 128


@functools.lru_cache(maxsize=None)
def _pos_encoding(seq_len: int, d: int):
    pos = np.arange(seq_len, dtype=np.float32)[:, None]
    i = np.arange(0, d, 2, dtype=np.float32)[None, :]
    angle = pos / np.power(10000.0, i / d)
    pe = np.zeros((seq_len, d), dtype=np.float32)
    pe[:, 0::2] = np.sin(angle)
    pe[:, 1::2] = np.cos(angle)
    return jnp.asarray(pe)


def _gcn_body(adj_ref, ids_ref, w_ref, b_ref, pe_ref, out_ref, h_ref):
    n_q = ids_ref.shape[0]
    embed = w_ref.shape[1]

    @pl.when(pl.program_id(0) == 0)
    def _():
        h_ref[...] = jnp.dot(ids_ref[...], w_ref[...],
                             preferred_element_type=jnp.float32)

    h = h_ref[...]
    a = adj_ref[...].astype(jnp.bfloat16).reshape(_CH, n_q, n_q)
    deg = jnp.sum(a, axis=1).astype(jnp.float32)
    dinv = jnp.where(deg > 0.0, jax.lax.rsqrt(jnp.maximum(deg, 1.0)), 0.0)
    dinv = dinv.astype(jnp.bfloat16)
    a_t = jnp.transpose(a, (0, 2, 1))
    c = a_t * dinv[:, :, None] * dinv[:, None, :]
    out = jnp.dot(c.reshape(_CH * n_q, n_q), h.astype(jnp.bfloat16),
                  preferred_element_type=jnp.float32)
    out = out + b_ref[...]
    out = out.reshape(_CH, n_q, embed) + pe_ref[...][:, None, :]
    out_ref[...] = out.reshape(_CH * n_q, embed)


def kernel(adj_matrices, ids, W, b):
    bsz, n_slices, n_q, _ = adj_matrices.shape
    embed = W.shape[1]
    num_blocks = bsz * n_slices
    assert num_blocks % _CH == 0 and n_slices % _CH == 0
    pe = _pos_encoding(n_slices, embed)
    adj2 = adj_matrices.reshape(num_blocks, n_q * n_q)
    b2 = b.reshape(1, embed)

    grid = (num_blocks // _CH,)
    out_flat = pl.pallas_call(
        _gcn_body,
        grid=grid,
        in_specs=[
            pl.BlockSpec((_CH, n_q * n_q), lambda g: (g, 0)),
            pl.BlockSpec((n_q, embed), lambda g: (0, 0)),
            pl.BlockSpec((embed, embed), lambda g: (0, 0)),
            pl.BlockSpec((1, embed), lambda g: (0, 0)),
            pl.BlockSpec((_CH, embed), lambda g: (g % (n_slices // _CH), 0)),
        ],
        out_specs=pl.BlockSpec((_CH * n_q, embed), lambda g: (g, 0)),
        out_shape=jax.ShapeDtypeStruct((num_blocks * n_q, embed), jnp.float32),
        scratch_shapes=[pltpu.VMEM((n_q, embed), jnp.float32)],
        compiler_params=pltpu.CompilerParams(
            dimension_semantics=("arbitrary",),
        ),
    )(adj2, ids[:n_q], W, b2, pe)
    return out_flat.reshape(bsz, n_slices, n_q, embed)
